# baseline (speedup 1.0000x reference)
import functools

import jax
import jax.numpy as jnp
from jax import lax
from jax.experimental import pallas as pl
from jax.experimental.pallas import tpu as pltpu
from jax.experimental.pallas import tpu_sc as plsc

B, L, D = 4096, 200, 64
NC, NS = 2, 16
NW = NC * NS
BPW = B // NW
CH0, CH1 = 104, 96


def _make_sc_call():
    mesh = plsc.VectorSubcoreMesh(core_axis_name="c", subcore_axis_name="s")

    @functools.partial(
        pl.kernel,
        out_type=jax.ShapeDtypeStruct((B, D), jnp.float32),
        mesh=mesh,
        compiler_params=pltpu.CompilerParams(use_tc_tiling_on_sc=False),
        scratch_types=[
            pltpu.VMEM((BPW * L,), jnp.int32),
            pltpu.VMEM((L, D), jnp.float32),
            pltpu.VMEM((L, D), jnp.float32),
            pltpu.VMEM((L, D), jnp.float32),
            pltpu.VMEM((L, D), jnp.float32),
            pltpu.VMEM((BPW, D), jnp.float32),
            pltpu.SemaphoreType.DMA,
            pltpu.SemaphoreType.DMA,
        ],
    )
    def sc_kernel(rid_hbm, e_hbm, tab_hbm, out_hbm,
                  idx_v, w0_v, w1_v, e0_v, e1_v, out_v, sem0, sem1):
        wid = lax.axis_index("s") * NC + lax.axis_index("c")
        base = wid * BPW

        pltpu.sync_copy(rid_hbm.at[wid], idx_v)

        def fetch(bl, w_v, e_v, sem):
            pltpu.async_copy(tab_hbm.at[idx_v.at[pl.ds(bl * L, CH0)]],
                             w_v.at[pl.ds(0, CH0)], sem)
            pltpu.async_copy(tab_hbm.at[idx_v.at[pl.ds(bl * L + CH0, CH1)]],
                             w_v.at[pl.ds(CH0, CH1)], sem)
            pltpu.async_copy(e_hbm.at[base + bl], e_v, sem)

        def wait(bl, w_v, e_v, sem):
            pltpu.make_async_copy(tab_hbm.at[idx_v.at[pl.ds(bl * L, CH0)]],
                                  w_v.at[pl.ds(0, CH0)], sem).wait()
            pltpu.make_async_copy(tab_hbm.at[idx_v.at[pl.ds(bl * L + CH0, CH1)]],
                                  w_v.at[pl.ds(CH0, CH1)], sem).wait()
            pltpu.make_async_copy(e_hbm.at[base + bl], e_v, sem).wait()

        lanes = lax.iota(jnp.int32, 16)
        perms = [(lanes ^ k)[:, None] for k in (8, 4, 2, 1)]
        _dnums = lax.GatherDimensionNumbers(
            offset_dims=(), collapsed_slice_dims=(0,), start_index_map=(0,))

        def hsum(x):
            for p in perms:
                x = x + lax.gather(
                    x, p, _dnums, (1,),
                    mode=lax.GatherScatterMode.PROMISE_IN_BOUNDS)
            return x

        UNROLL = 4
        assert L % UNROLL == 0

        def compute(bl, w_v, e_v):
            def body(i, accs):
                a0, a1, a2, a3 = accs
                lb = i * UNROLL
                for u in range(UNROLL):
                    l = lb + u
                    w0 = w_v[l, pl.ds(0, 16)]
                    w1 = w_v[l, pl.ds(16, 16)]
                    w2 = w_v[l, pl.ds(32, 16)]
                    w3 = w_v[l, pl.ds(48, 16)]
                    e0 = e_v[l, pl.ds(0, 16)]
                    e1 = e_v[l, pl.ds(16, 16)]
                    e2 = e_v[l, pl.ds(32, 16)]
                    e3 = e_v[l, pl.ds(48, 16)]
                    nsq = w0 * w0 + w1 * w1 + w2 * w2 + w3 * w3
                    dot = e0 * w0 + e1 * w1 + e2 * w2 + e3 * w3
                    ns = hsum(nsq)
                    dt = hsum(dot)
                    c = dt / jnp.maximum(ns, jnp.float32(1e-24))
                    a0 = a0 + (e0 - c * w0)
                    a1 = a1 + (e1 - c * w1)
                    a2 = a2 + (e2 - c * w2)
                    a3 = a3 + (e3 - c * w3)
                return (a0, a1, a2, a3)

            z = jnp.zeros((16,), jnp.float32)
            a0, a1, a2, a3 = lax.fori_loop(0, L // UNROLL, body, (z, z, z, z))
            scale = jnp.float32(L)
            out_v[bl, pl.ds(0, 16)] = a0 / scale
            out_v[bl, pl.ds(16, 16)] = a1 / scale
            out_v[bl, pl.ds(32, 16)] = a2 / scale
            out_v[bl, pl.ds(48, 16)] = a3 / scale

        fetch(0, w0_v, e0_v, sem0)

        def pair(j, carry):
            b0 = 2 * j
            fetch(b0 + 1, w1_v, e1_v, sem1)
            wait(b0, w0_v, e0_v, sem0)
            compute(b0, w0_v, e0_v)

            @pl.when(j < BPW // 2 - 1)
            def _():
                fetch(b0 + 2, w0_v, e0_v, sem0)

            wait(b0 + 1, w1_v, e1_v, sem1)
            compute(b0 + 1, w1_v, e1_v)
            return carry

        lax.fori_loop(0, BPW // 2, pair, 0)
        pltpu.sync_copy(out_v, out_hbm.at[pl.ds(base, BPW)])

    return sc_kernel


_sc_call = _make_sc_call()


def kernel(batch_nei_rid, batch_nei_e_emb, w_r_table):
    rid_flat = batch_nei_rid.reshape(NW, BPW * L)
    return _sc_call(rid_flat, batch_nei_e_emb, w_r_table)


# speedup vs baseline: 1.1738x; 1.1738x over previous
import functools

import jax
import jax.numpy as jnp
from jax import lax
from jax.experimental import pallas as pl
from jax.experimental.pallas import tpu as pltpu
from jax.experimental.pallas import tpu_sc as plsc

B, L, D = 4096, 200, 64
DP = 128
NC, NS = 2, 16
NW = NC * NS
BPW = B // NW
HB = BPW // 2
CH0, CH1 = 104, 96


def _make_sc_call():
    mesh = plsc.VectorSubcoreMesh(core_axis_name="c", subcore_axis_name="s")

    @functools.partial(
        pl.kernel,
        out_type=jax.ShapeDtypeStruct((B * D,), jnp.float32),
        mesh=mesh,
        compiler_params=pltpu.CompilerParams(use_tc_tiling_on_sc=True),
        scratch_types=[
            pltpu.VMEM((HB * L,), jnp.int32),
            pltpu.VMEM((L, DP), jnp.float32),
            pltpu.VMEM((L, DP), jnp.float32),
            pltpu.VMEM((L, D), jnp.float32),
            pltpu.VMEM((L, D), jnp.float32),
            pltpu.VMEM((BPW * D,), jnp.float32),
            pltpu.SemaphoreType.DMA,
            pltpu.SemaphoreType.DMA,
        ],
    )
    def sc_kernel(rid_hbm, e_hbm, tab_hbm, out_hbm,
                  idx_v, w0_v, w1_v, e0_v, e1_v, out_v, sem0, sem1):
        wid = lax.axis_index("s") * NC + lax.axis_index("c")
        base = wid * BPW

        def load_idx(half):
            pltpu.sync_copy(
                rid_hbm.at[pl.ds((base + half * HB) * L, HB * L)], idx_v)

        def fetch(bi, bg, w_v, e_v, sem):
            pltpu.async_copy(tab_hbm.at[idx_v.at[pl.ds(bi * L, CH0)]],
                             w_v.at[pl.ds(0, CH0)], sem)
            pltpu.async_copy(tab_hbm.at[idx_v.at[pl.ds(bi * L + CH0, CH1)]],
                             w_v.at[pl.ds(CH0, CH1)], sem)
            pltpu.async_copy(e_hbm.at[base + bg], e_v, sem)

        def wait(bi, bg, w_v, e_v, sem):
            pltpu.make_async_copy(tab_hbm.at[idx_v.at[pl.ds(bi * L, CH0)]],
                                  w_v.at[pl.ds(0, CH0)], sem).wait()
            pltpu.make_async_copy(tab_hbm.at[idx_v.at[pl.ds(bi * L + CH0, CH1)]],
                                  w_v.at[pl.ds(CH0, CH1)], sem).wait()
            pltpu.make_async_copy(e_hbm.at[base + bg], e_v, sem).wait()

        lanes = lax.iota(jnp.int32, 16)
        perms = [(lanes ^ k)[:, None] for k in (8, 4, 2, 1)]
        _dnums = lax.GatherDimensionNumbers(
            offset_dims=(), collapsed_slice_dims=(0,), start_index_map=(0,))

        def hsum(x):
            for p in perms:
                x = x + lax.gather(
                    x, p, _dnums, (1,),
                    mode=lax.GatherScatterMode.PROMISE_IN_BOUNDS)
            return x

        UNROLL = 4
        assert L % UNROLL == 0

        def compute(bl, w_v, e_v):
            def body(i, accs):
                a0, a1, a2, a3 = accs
                lb = i * UNROLL
                for u in range(UNROLL):
                    l = lb + u
                    w0 = w_v[l, pl.ds(0, 16)]
                    w1 = w_v[l, pl.ds(16, 16)]
                    w2 = w_v[l, pl.ds(32, 16)]
                    w3 = w_v[l, pl.ds(48, 16)]
                    e0 = e_v[l, pl.ds(0, 16)]
                    e1 = e_v[l, pl.ds(16, 16)]
                    e2 = e_v[l, pl.ds(32, 16)]
                    e3 = e_v[l, pl.ds(48, 16)]
                    nsq = w0 * w0 + w1 * w1 + w2 * w2 + w3 * w3
                    dot = e0 * w0 + e1 * w1 + e2 * w2 + e3 * w3
                    ns = hsum(nsq)
                    dt = hsum(dot)
                    c = dt / jnp.maximum(ns, jnp.float32(1e-24))
                    a0 = a0 + (e0 - c * w0)
                    a1 = a1 + (e1 - c * w1)
                    a2 = a2 + (e2 - c * w2)
                    a3 = a3 + (e3 - c * w3)
                return (a0, a1, a2, a3)

            z = jnp.zeros((16,), jnp.float32)
            a0, a1, a2, a3 = lax.fori_loop(0, L // UNROLL, body, (z, z, z, z))
            scale = jnp.float32(L)
            ob = bl * D
            out_v[pl.ds(ob, 16)] = a0 / scale
            out_v[pl.ds(ob + 16, 16)] = a1 / scale
            out_v[pl.ds(ob + 32, 16)] = a2 / scale
            out_v[pl.ds(ob + 48, 16)] = a3 / scale

        def half_loop(half):
            hb0 = half * HB
            fetch(0, hb0, w0_v, e0_v, sem0)

            def pair(j, carry):
                bi = 2 * j
                bg = hb0 + bi
                fetch(bi + 1, bg + 1, w1_v, e1_v, sem1)
                wait(bi, bg, w0_v, e0_v, sem0)
                compute(bg, w0_v, e0_v)

                @pl.when(j < HB // 2 - 1)
                def _():
                    fetch(bi + 2, bg + 2, w0_v, e0_v, sem0)

                wait(bi + 1, bg + 1, w1_v, e1_v, sem1)
                compute(bg + 1, w1_v, e1_v)
                return carry

            lax.fori_loop(0, HB // 2, pair, 0)

        load_idx(0)
        half_loop(0)
        load_idx(1)
        half_loop(1)
        pltpu.sync_copy(out_v, out_hbm.at[pl.ds(base * D, BPW * D)])

    return sc_kernel


_sc_call = _make_sc_call()


def kernel(batch_nei_rid, batch_nei_e_emb, w_r_table):
    rid_flat = batch_nei_rid.reshape(B * L)
    tab_pad = jnp.pad(w_r_table, ((0, 0), (0, DP - D)))
    out = _sc_call(rid_flat, batch_nei_e_emb, tab_pad)
    return out.reshape(B, D)
